# baseline (speedup 1.0000x reference)
import jax
import jax.numpy as jnp
from jax.experimental import pallas as pl
from jax.experimental.pallas import tpu as pltpu

N_PIX = 262144
K = 1024
BLOCK = 1024


def _vq_body(xa_ref, w_ref, ckc_ref, o_ref):
    d = jnp.dot(xa_ref[...], w_ref[...],
                preferred_element_type=jnp.float32)
    idx = jnp.argmin(d, axis=1)
    onehot = (jax.lax.broadcasted_iota(jnp.int32, (BLOCK, K), 1)
              == idx[:, None]).astype(jnp.float32)
    o_ref[...] = jnp.dot(onehot, ckc_ref[...],
                         preferred_element_type=jnp.float32)


def kernel(x, cluster_centers):
    xa = jnp.concatenate(
        [x, jnp.ones((N_PIX, 1), jnp.float32)], axis=1)
    ccsq = jnp.sum(cluster_centers * cluster_centers, axis=1)
    w = jnp.concatenate(
        [-2.0 * cluster_centers.T, ccsq[None, :]], axis=0)
    grid = (N_PIX // BLOCK,)
    return pl.pallas_call(
        _vq_body,
        grid=grid,
        in_specs=[
            pl.BlockSpec((BLOCK, 4), lambda i: (i, 0)),
            pl.BlockSpec((4, K), lambda i: (0, 0)),
            pl.BlockSpec((K, 3), lambda i: (0, 0)),
        ],
        out_specs=pl.BlockSpec((BLOCK, 3), lambda i: (i, 0)),
        out_shape=jax.ShapeDtypeStruct((N_PIX, 3), jnp.float32),
        compiler_params=pltpu.CompilerParams(
            dimension_semantics=("arbitrary",),
        ),
    )(xa, w, cluster_centers)


# speedup vs baseline: 1.0851x; 1.0851x over previous
import jax
import jax.numpy as jnp
from jax.experimental import pallas as pl
from jax.experimental.pallas import tpu as pltpu

N_PIX = 262144
K = 1024
BLOCK = 1024


def _vq_body(x_ref, ct_ref, ckc_ref, o_ref):
    x0 = x_ref[:, 0:1]
    x1 = x_ref[:, 1:2]
    x2 = x_ref[:, 2:3]
    d0 = x0 - ct_ref[0:1, :]
    d1 = x1 - ct_ref[1:2, :]
    d2 = x2 - ct_ref[2:3, :]
    d = d0 * d0 + d1 * d1 + d2 * d2
    m = jnp.min(d, axis=1, keepdims=True)
    onehot = (d <= m).astype(jnp.float32)
    g = jnp.dot(onehot, ckc_ref[...],
                preferred_element_type=jnp.float32)
    o_ref[...] = g[:, 0:3] / g[:, 3:4]


def kernel(x, cluster_centers):
    ct = cluster_centers.T
    ckc = jnp.concatenate(
        [cluster_centers, jnp.ones((K, 1), jnp.float32)], axis=1)
    grid = (N_PIX // BLOCK,)
    return pl.pallas_call(
        _vq_body,
        grid=grid,
        in_specs=[
            pl.BlockSpec((BLOCK, 3), lambda i: (i, 0)),
            pl.BlockSpec((3, K), lambda i: (0, 0)),
            pl.BlockSpec((K, 4), lambda i: (0, 0)),
        ],
        out_specs=pl.BlockSpec((BLOCK, 3), lambda i: (i, 0)),
        out_shape=jax.ShapeDtypeStruct((N_PIX, 3), jnp.float32),
        compiler_params=pltpu.CompilerParams(
            dimension_semantics=("arbitrary",),
        ),
    )(x, ct, ckc)


# speedup vs baseline: 1.1103x; 1.0233x over previous
import jax
import jax.numpy as jnp
from jax.experimental import pallas as pl
from jax.experimental.pallas import tpu as pltpu

N_PIX = 262144
K = 1024
BLOCK = 1024


def _vq_body(x_ref, w_ref, ckc_ref, o_ref):
    x0 = x_ref[:, 0:1]
    x1 = x_ref[:, 1:2]
    x2 = x_ref[:, 2:3]
    d = ((x0 * w_ref[0:1, :] + w_ref[3:4, :])
         + (x1 * w_ref[1:2, :] + x2 * w_ref[2:3, :]))
    m = jnp.min(d, axis=1, keepdims=True)
    onehot = (d <= m).astype(jnp.float32)
    g = jnp.dot(onehot, ckc_ref[...],
                preferred_element_type=jnp.float32)
    o_ref[...] = g[:, 0:3] / g[:, 3:4]


def kernel(x, cluster_centers):
    ccsq = jnp.sum(cluster_centers * cluster_centers, axis=1)
    w = jnp.concatenate(
        [-2.0 * cluster_centers.T, ccsq[None, :]], axis=0)
    ckc = jnp.concatenate(
        [cluster_centers, jnp.ones((K, 1), jnp.float32)], axis=1)
    grid = (N_PIX // BLOCK,)
    return pl.pallas_call(
        _vq_body,
        grid=grid,
        in_specs=[
            pl.BlockSpec((BLOCK, 3), lambda i: (i, 0)),
            pl.BlockSpec((4, K), lambda i: (0, 0)),
            pl.BlockSpec((K, 4), lambda i: (0, 0)),
        ],
        out_specs=pl.BlockSpec((BLOCK, 3), lambda i: (i, 0)),
        out_shape=jax.ShapeDtypeStruct((N_PIX, 3), jnp.float32),
        compiler_params=pltpu.CompilerParams(
            dimension_semantics=("arbitrary",),
        ),
    )(x, w, ckc)


# speedup vs baseline: 1.2362x; 1.1133x over previous
import jax
import jax.numpy as jnp
from jax.experimental import pallas as pl
from jax.experimental.pallas import tpu as pltpu

N_PIX = 262144
K = 1024
BLOCK = 2048


def _vq_body(x_ref, w_ref, ckc_ref, o_ref):
    x0 = x_ref[:, 0:1]
    x1 = x_ref[:, 1:2]
    x2 = x_ref[:, 2:3]
    d = ((x0 * w_ref[0:1, :] + w_ref[3:4, :])
         + (x1 * w_ref[1:2, :] + x2 * w_ref[2:3, :]))
    m = jnp.min(d, axis=1, keepdims=True)
    onehot = (d <= m).astype(jnp.float32)
    g = jnp.dot(onehot, ckc_ref[...],
                preferred_element_type=jnp.float32)
    o_ref[...] = g[:, 0:3] / g[:, 3:4]


def kernel(x, cluster_centers):
    ccsq = jnp.sum(cluster_centers * cluster_centers, axis=1)
    w = jnp.concatenate(
        [-2.0 * cluster_centers.T, ccsq[None, :]], axis=0)
    ckc = jnp.concatenate(
        [cluster_centers, jnp.ones((K, 1), jnp.float32)], axis=1)
    grid = (N_PIX // BLOCK,)
    return pl.pallas_call(
        _vq_body,
        grid=grid,
        in_specs=[
            pl.BlockSpec((BLOCK, 3), lambda i: (i, 0)),
            pl.BlockSpec((4, K), lambda i: (0, 0)),
            pl.BlockSpec((K, 4), lambda i: (0, 0)),
        ],
        out_specs=pl.BlockSpec((BLOCK, 3), lambda i: (i, 0)),
        out_shape=jax.ShapeDtypeStruct((N_PIX, 3), jnp.float32),
        compiler_params=pltpu.CompilerParams(
            dimension_semantics=("arbitrary",),
        ),
    )(x, w, ckc)


# speedup vs baseline: 1.4263x; 1.1538x over previous
import jax
import jax.numpy as jnp
from jax.experimental import pallas as pl
from jax.experimental.pallas import tpu as pltpu

N_PIX = 262144
K = 1024
BLOCK = 4096


def _vq_body(x_ref, w_ref, ckc_ref, o_ref):
    x0 = x_ref[:, 0:1]
    x1 = x_ref[:, 1:2]
    x2 = x_ref[:, 2:3]
    d = ((x0 * w_ref[0:1, :] + w_ref[3:4, :])
         + (x1 * w_ref[1:2, :] + x2 * w_ref[2:3, :]))
    m = jnp.min(d, axis=1, keepdims=True)
    onehot = (d <= m).astype(jnp.float32)
    g = jnp.dot(onehot, ckc_ref[...],
                preferred_element_type=jnp.float32)
    o_ref[...] = g[:, 0:3] / g[:, 3:4]


def kernel(x, cluster_centers):
    ccsq = jnp.sum(cluster_centers * cluster_centers, axis=1)
    w = jnp.concatenate(
        [-2.0 * cluster_centers.T, ccsq[None, :]], axis=0)
    ckc = jnp.concatenate(
        [cluster_centers, jnp.ones((K, 1), jnp.float32)], axis=1)
    grid = (N_PIX // BLOCK,)
    return pl.pallas_call(
        _vq_body,
        grid=grid,
        in_specs=[
            pl.BlockSpec((BLOCK, 3), lambda i: (i, 0)),
            pl.BlockSpec((4, K), lambda i: (0, 0)),
            pl.BlockSpec((K, 4), lambda i: (0, 0)),
        ],
        out_specs=pl.BlockSpec((BLOCK, 3), lambda i: (i, 0)),
        out_shape=jax.ShapeDtypeStruct((N_PIX, 3), jnp.float32),
        compiler_params=pltpu.CompilerParams(
            dimension_semantics=("arbitrary",),
        ),
    )(x, w, ckc)


# speedup vs baseline: 1.4485x; 1.0156x over previous
import jax
import jax.numpy as jnp
from jax.experimental import pallas as pl
from jax.experimental.pallas import tpu as pltpu

N_PIX = 262144
K = 1024
BLOCK = 8192


def _vq_body(x_ref, w_ref, ckc_ref, o_ref):
    x0 = x_ref[:, 0:1]
    x1 = x_ref[:, 1:2]
    x2 = x_ref[:, 2:3]
    d = ((x0 * w_ref[0:1, :] + w_ref[3:4, :])
         + (x1 * w_ref[1:2, :] + x2 * w_ref[2:3, :]))
    m = jnp.min(d, axis=1, keepdims=True)
    onehot = (d <= m).astype(jnp.float32)
    g = jnp.dot(onehot, ckc_ref[...],
                preferred_element_type=jnp.float32)
    o_ref[...] = g[:, 0:3] / g[:, 3:4]


def kernel(x, cluster_centers):
    ccsq = jnp.sum(cluster_centers * cluster_centers, axis=1)
    w = jnp.concatenate(
        [-2.0 * cluster_centers.T, ccsq[None, :]], axis=0)
    ckc = jnp.concatenate(
        [cluster_centers, jnp.ones((K, 1), jnp.float32)], axis=1)
    grid = (N_PIX // BLOCK,)
    return pl.pallas_call(
        _vq_body,
        grid=grid,
        in_specs=[
            pl.BlockSpec((BLOCK, 3), lambda i: (i, 0)),
            pl.BlockSpec((4, K), lambda i: (0, 0)),
            pl.BlockSpec((K, 4), lambda i: (0, 0)),
        ],
        out_specs=pl.BlockSpec((BLOCK, 3), lambda i: (i, 0)),
        out_shape=jax.ShapeDtypeStruct((N_PIX, 3), jnp.float32),
        compiler_params=pltpu.CompilerParams(
            dimension_semantics=("arbitrary",),
        ),
    )(x, w, ckc)


# speedup vs baseline: 2.3353x; 1.6122x over previous
import jax
import jax.numpy as jnp
from jax.experimental import pallas as pl
from jax.experimental.pallas import tpu as pltpu

N_PIX = 262144
K = 1024
BLOCK = 8192


def _vq_body(xt_ref, wt_ref, ckct_ref, o_ref):
    x0 = xt_ref[0:1, :]
    x1 = xt_ref[1:2, :]
    x2 = xt_ref[2:3, :]
    d = ((wt_ref[:, 0:1] * x0 + wt_ref[:, 3:4])
         + (wt_ref[:, 1:2] * x1 + wt_ref[:, 2:3] * x2))
    m = jnp.min(d, axis=0, keepdims=True)
    onehot = (d <= m).astype(jnp.float32)
    g = jnp.dot(ckct_ref[...], onehot,
                preferred_element_type=jnp.float32)
    o_ref[...] = g[0:3, :] / g[3:4, :]


def kernel(x, cluster_centers):
    ccsq = jnp.sum(cluster_centers * cluster_centers, axis=1)
    wt = jnp.concatenate(
        [-2.0 * cluster_centers, ccsq[:, None]], axis=1)
    ckct = jnp.concatenate(
        [cluster_centers.T, jnp.ones((1, K), jnp.float32)], axis=0)
    xt = x.T
    grid = (N_PIX // BLOCK,)
    out_t = pl.pallas_call(
        _vq_body,
        grid=grid,
        in_specs=[
            pl.BlockSpec((3, BLOCK), lambda i: (0, i)),
            pl.BlockSpec((K, 4), lambda i: (0, 0)),
            pl.BlockSpec((4, K), lambda i: (0, 0)),
        ],
        out_specs=pl.BlockSpec((3, BLOCK), lambda i: (0, i)),
        out_shape=jax.ShapeDtypeStruct((3, N_PIX), jnp.float32),
        compiler_params=pltpu.CompilerParams(
            dimension_semantics=("arbitrary",),
        ),
    )(xt, wt, ckct)
    return out_t.T
